# baseline (speedup 1.0000x reference)
import dataclasses

import jax
import jax.numpy as jnp
from jax import lax
from jax.experimental import pallas as pl
from jax.experimental.pallas import tpu as pltpu
from jax.experimental.pallas import tpu_sc as plsc

N, E, D = 10000, 320000, 128

NC = 2
NS = 16
NW = NC * NS
EPW = E // NW
NST = 10
B = 100
NJS = EPW // (NST * B)
EPS = NJS * B
RPS = 632
NP = NS * RPS


def _linear_body(x_ref, w_ref, b_ref, o_ref):
    o_ref[...] = lax.dot_general(
        x_ref[...], w_ref[...],
        dimension_numbers=(((1,), (1,)), ((), ())),
        preferred_element_type=jnp.float32,
    ) + b_ref[...]


def _linear(x, W, b):
    blk = 1000
    return pl.pallas_call(
        _linear_body,
        grid=(N // blk,),
        in_specs=[
            pl.BlockSpec((blk, D), lambda i: (i, 0)),
            pl.BlockSpec((D, D), lambda i: (0, 0)),
            pl.BlockSpec((1, D), lambda i: (0, 0)),
        ],
        out_specs=pl.BlockSpec((blk, D), lambda i: (i, 0)),
        out_shape=jax.ShapeDtypeStruct((N, D), jnp.float32),
    )(x, W, b.reshape(1, D))


def _sc_body(support_hbm, src_hbm, dst_hbm, w_hbm, zeros_hbm, out_hbm,
             src0, src1, dst0, dst1, w0, w1, gbuf0, gbuf1, gbuf2, acc,
             sg0, sg1, sg2, ss0, ss1, ss2, sem_st, sem_i):
    cid = lax.axis_index("c")
    sid = lax.axis_index("s")
    wid = sid * NC + cid
    srcbufs = (src0, src1)
    dstbufs = (dst0, dst1)
    wbufs = (w0, w1)
    gbufs = (gbuf0, gbuf1, gbuf2)
    gsems = (sg0, sg1, sg2)
    ssems = (ss0, ss1, ss2)

    def stage_in(s, bufset):
        pltpu.async_copy(src_hbm.at[wid, s], srcbufs[bufset], sem_st)
        pltpu.async_copy(dst_hbm.at[wid, s], dstbufs[bufset], sem_st)
        pltpu.async_copy(w_hbm.at[pl.ds(wid * EPW + s * EPS, EPS)],
                         wbufs[bufset], sem_st)

    def stage_wait(s, bufset):
        pltpu.make_async_copy(src_hbm.at[wid, s], srcbufs[bufset],
                              sem_st).wait()
        pltpu.make_async_copy(dst_hbm.at[wid, s], dstbufs[bufset],
                              sem_st).wait()
        pltpu.make_async_copy(w_hbm.at[pl.ds(wid * EPW + s * EPS, EPS)],
                              wbufs[bufset], sem_st).wait()

    stage_in(0, 0)
    pltpu.async_copy(zeros_hbm.at[pl.ds(sid * RPS, RPS)],
                     acc.at[pl.ds(sid * RPS, RPS)], sem_i).wait()
    plsc.subcore_barrier()
    stage_wait(0, 0)

    pltpu.async_copy(support_hbm.at[src0.at[0]], gbuf0, sg0)
    pltpu.async_copy(support_hbm.at[src0.at[1]], gbuf1, sg1)

    def _step(jj, m, srcbuf, dstbuf, wbuf):
        gbuf = gbufs[m]
        q = (m + 2) % 3
        pltpu.make_async_copy(support_hbm.at[srcbuf.at[jj]], gbuf,
                              gsems[m]).wait()

        @plsc.parallel_loop(0, B, unroll=8)
        def _(e):
            wsplat = plsc.load_gather(
                wbuf, [jnp.full((16,), jj * B + e, jnp.int32)])
            for c in range(D // 16):
                sl = pl.ds(c * 16, 16)
                gbuf[e, sl] = gbuf[e, sl] * wsplat

        pltpu.async_copy(gbuf, acc.at[dstbuf.at[jj]], ssems[m], add=True)

        @pl.when(jj + 2 < NJS)
        def _():
            @pl.when(jj >= 1)
            def _():
                pltpu.make_async_copy(gbufs[q], acc.at[dstbuf.at[jj]],
                                      ssems[q]).wait()
            pltpu.async_copy(support_hbm.at[srcbuf.at[jj + 2]],
                             gbufs[q], gsems[q])

    for s in range(NST):
        cs = s % 2
        srcbuf, dstbuf, wbuf = srcbufs[cs], dstbufs[cs], wbufs[cs]
        if s + 1 < NST:
            stage_in(s + 1, 1 - cs)

        @pl.loop(0, NJS - 1, step=3)
        def _(j):
            for m in range(3):
                _step(j + m, m, srcbuf, dstbuf, wbuf)

        _step(NJS - 1, (NJS - 1) % 3, srcbuf, dstbuf, wbuf)

        for i in range(3):
            jj = NJS - 3 + i
            pltpu.make_async_copy(gbufs[jj % 3], acc.at[dstbuf.at[jj]],
                                  ssems[jj % 3]).wait()

        if s + 1 < NST:
            stage_wait(s + 1, 1 - cs)
            nsrc = srcbufs[1 - cs]
            pltpu.async_copy(support_hbm.at[nsrc.at[0]], gbuf0, sg0)
            pltpu.async_copy(support_hbm.at[nsrc.at[1]], gbuf1, sg1)

    plsc.subcore_barrier()
    pltpu.sync_copy(acc.at[pl.ds(sid * RPS, RPS)],
                    out_hbm.at[cid, pl.ds(sid * RPS, RPS)])


def _sc_scatter(support, src, dst, w, zeros):
    mesh = plsc.VectorSubcoreMesh(core_axis_name="c", subcore_axis_name="s")
    cp = pltpu.CompilerParams()
    if "needs_layout_passes" in pltpu.CompilerParams.__dataclass_fields__:
        cp = dataclasses.replace(cp, needs_layout_passes=False)
    kern = pl.kernel(
        _sc_body,
        compiler_params=cp,
        out_type=jax.ShapeDtypeStruct((NC, NP, D), jnp.float32),
        mesh=mesh,
        scratch_types=[
            pltpu.VMEM((NJS, B), jnp.int32),
            pltpu.VMEM((NJS, B), jnp.int32),
            pltpu.VMEM((NJS, B), jnp.int32),
            pltpu.VMEM((NJS, B), jnp.int32),
            pltpu.VMEM((EPS,), jnp.float32),
            pltpu.VMEM((EPS,), jnp.float32),
            pltpu.VMEM((B, D), jnp.float32),
            pltpu.VMEM((B, D), jnp.float32),
            pltpu.VMEM((B, D), jnp.float32),
            pltpu.VMEM_SHARED((NP, D), jnp.float32),
            pltpu.SemaphoreType.DMA,
            pltpu.SemaphoreType.DMA,
            pltpu.SemaphoreType.DMA,
            pltpu.SemaphoreType.DMA,
            pltpu.SemaphoreType.DMA,
            pltpu.SemaphoreType.DMA,
            pltpu.SemaphoreType.DMA,
            pltpu.SemaphoreType.DMA,
        ],
    )
    return kern(support, src, dst, w, zeros)


def _combine_body(p_ref, o_ref):
    o_ref[...] = p_ref[0] + p_ref[1]


def _combine(partials):
    blk = 1000
    return pl.pallas_call(
        _combine_body,
        grid=(N // blk,),
        in_specs=[pl.BlockSpec((NC, blk, D), lambda i: (0, i, 0))],
        out_specs=pl.BlockSpec((blk, D), lambda i: (i, 0)),
        out_shape=jax.ShapeDtypeStruct((N, D), jnp.float32),
    )(partials)


@jax.jit
def _impl(x, edge_index, edge_weight, W, b):
    support = _linear(x, W, b)
    src = edge_index[0].astype(jnp.int32).reshape(NW, NST, NJS, B)
    dst = edge_index[1].astype(jnp.int32).reshape(NW, NST, NJS, B)
    zeros = jnp.zeros((NP, D), jnp.float32)
    partials = _sc_scatter(support, src, dst, edge_weight, zeros)
    return (partials[0, :N] + partials[1, :N])


def kernel(x, edge_index, edge_weight, W, b):
    return _impl(x, edge_index, edge_weight, W, b)


# speedup vs baseline: 1.0208x; 1.0208x over previous
import dataclasses

import jax
import jax.numpy as jnp
from jax import lax
from jax.experimental import pallas as pl
from jax.experimental.pallas import tpu as pltpu
from jax.experimental.pallas import tpu_sc as plsc

N, E, D = 10000, 320000, 128

NC = 2
NS = 16
NW = NC * NS
EPW = E // NW
NST = 10
B = 100
NJS = EPW // (NST * B)
EPS = NJS * B
RPS = 632
NP = NS * RPS


def _linear_body(x_ref, w_ref, b_ref, o_ref):
    o_ref[...] = lax.dot_general(
        x_ref[...], w_ref[...],
        dimension_numbers=(((1,), (1,)), ((), ())),
        preferred_element_type=jnp.float32,
    ) + b_ref[...]


def _linear(x, W, b):
    blk = 1000
    return pl.pallas_call(
        _linear_body,
        grid=(N // blk,),
        in_specs=[
            pl.BlockSpec((blk, D), lambda i: (i, 0)),
            pl.BlockSpec((D, D), lambda i: (0, 0)),
            pl.BlockSpec((1, D), lambda i: (0, 0)),
        ],
        out_specs=pl.BlockSpec((blk, D), lambda i: (i, 0)),
        out_shape=jax.ShapeDtypeStruct((N, D), jnp.float32),
    )(x, W, b.reshape(1, D))


def _sc_body(support_hbm, src_hbm, dst_hbm, w_hbm, zeros_hbm, out_hbm,
             src0, src1, dst0, dst1, w0, w1, gbuf0, gbuf1, gbuf2, acc,
             sg0, sg1, sg2, ss0, ss1, ss2, sem_st, sem_i):
    cid = lax.axis_index("c")
    sid = lax.axis_index("s")
    wid = sid * NC + cid
    srcbufs = (src0, src1)
    dstbufs = (dst0, dst1)
    wbufs = (w0, w1)
    gbufs = (gbuf0, gbuf1, gbuf2)
    gsems = (sg0, sg1, sg2)
    ssems = (ss0, ss1, ss2)

    def stage_in(s, bufset):
        pltpu.async_copy(src_hbm.at[wid, s], srcbufs[bufset], sem_st)
        pltpu.async_copy(dst_hbm.at[wid, s], dstbufs[bufset], sem_st)
        pltpu.async_copy(w_hbm.at[pl.ds(wid * EPW + s * EPS, EPS)],
                         wbufs[bufset], sem_st)

    def stage_wait(s, bufset):
        pltpu.make_async_copy(src_hbm.at[wid, s], srcbufs[bufset],
                              sem_st).wait()
        pltpu.make_async_copy(dst_hbm.at[wid, s], dstbufs[bufset],
                              sem_st).wait()
        pltpu.make_async_copy(w_hbm.at[pl.ds(wid * EPW + s * EPS, EPS)],
                              wbufs[bufset], sem_st).wait()

    stage_in(0, 0)
    pltpu.async_copy(zeros_hbm.at[pl.ds(sid * RPS, RPS)],
                     acc.at[pl.ds(sid * RPS, RPS)], sem_i).wait()
    plsc.subcore_barrier()
    stage_wait(0, 0)

    pltpu.async_copy(support_hbm.at[src0.at[0]], gbuf0, sg0)
    pltpu.async_copy(support_hbm.at[src0.at[1]], gbuf1, sg1)

    def _step(jj, m, srcbuf, dstbuf, wbuf):
        gbuf = gbufs[m]
        q = (m + 2) % 3
        pltpu.make_async_copy(support_hbm.at[srcbuf.at[jj]], gbuf,
                              gsems[m]).wait()

        @pl.when(jj + 2 < NJS)
        def _():
            @pl.when(jj >= 1)
            def _():
                pltpu.make_async_copy(gbufs[q], acc.at[dstbuf.at[jj]],
                                      ssems[q]).wait()
            pltpu.async_copy(support_hbm.at[srcbuf.at[jj + 2]],
                             gbufs[q], gsems[q])

        @plsc.parallel_loop(0, B, unroll=4)
        def _(e):
            wsplat = plsc.load_gather(
                wbuf, [jnp.full((16,), jj * B + e, jnp.int32)])
            for c in range(D // 16):
                sl = pl.ds(c * 16, 16)
                gbuf[e, sl] = gbuf[e, sl] * wsplat

        pltpu.async_copy(gbuf, acc.at[dstbuf.at[jj]], ssems[m], add=True)

    for s in range(NST):
        cs = s % 2
        srcbuf, dstbuf, wbuf = srcbufs[cs], dstbufs[cs], wbufs[cs]
        if s + 1 < NST:
            stage_in(s + 1, 1 - cs)

        @pl.loop(0, NJS - 1, step=3)
        def _(j):
            for m in range(3):
                _step(j + m, m, srcbuf, dstbuf, wbuf)

        _step(NJS - 1, (NJS - 1) % 3, srcbuf, dstbuf, wbuf)

        for i in range(3):
            jj = NJS - 3 + i
            pltpu.make_async_copy(gbufs[jj % 3], acc.at[dstbuf.at[jj]],
                                  ssems[jj % 3]).wait()

        if s + 1 < NST:
            stage_wait(s + 1, 1 - cs)
            nsrc = srcbufs[1 - cs]
            pltpu.async_copy(support_hbm.at[nsrc.at[0]], gbuf0, sg0)
            pltpu.async_copy(support_hbm.at[nsrc.at[1]], gbuf1, sg1)

    plsc.subcore_barrier()
    pltpu.sync_copy(acc.at[pl.ds(sid * RPS, RPS)],
                    out_hbm.at[cid, pl.ds(sid * RPS, RPS)])


def _sc_scatter(support, src, dst, w, zeros):
    mesh = plsc.VectorSubcoreMesh(core_axis_name="c", subcore_axis_name="s")
    cp = pltpu.CompilerParams()
    if "needs_layout_passes" in pltpu.CompilerParams.__dataclass_fields__:
        cp = dataclasses.replace(cp, needs_layout_passes=False)
    kern = pl.kernel(
        _sc_body,
        compiler_params=cp,
        out_type=jax.ShapeDtypeStruct((NC, NP, D), jnp.float32),
        mesh=mesh,
        scratch_types=[
            pltpu.VMEM((NJS, B), jnp.int32),
            pltpu.VMEM((NJS, B), jnp.int32),
            pltpu.VMEM((NJS, B), jnp.int32),
            pltpu.VMEM((NJS, B), jnp.int32),
            pltpu.VMEM((EPS,), jnp.float32),
            pltpu.VMEM((EPS,), jnp.float32),
            pltpu.VMEM((B, D), jnp.float32),
            pltpu.VMEM((B, D), jnp.float32),
            pltpu.VMEM((B, D), jnp.float32),
            pltpu.VMEM_SHARED((NP, D), jnp.float32),
            pltpu.SemaphoreType.DMA,
            pltpu.SemaphoreType.DMA,
            pltpu.SemaphoreType.DMA,
            pltpu.SemaphoreType.DMA,
            pltpu.SemaphoreType.DMA,
            pltpu.SemaphoreType.DMA,
            pltpu.SemaphoreType.DMA,
            pltpu.SemaphoreType.DMA,
        ],
    )
    return kern(support, src, dst, w, zeros)


def _combine_body(p_ref, o_ref):
    o_ref[...] = p_ref[0] + p_ref[1]


def _combine(partials):
    blk = 1000
    return pl.pallas_call(
        _combine_body,
        grid=(N // blk,),
        in_specs=[pl.BlockSpec((NC, blk, D), lambda i: (0, i, 0))],
        out_specs=pl.BlockSpec((blk, D), lambda i: (i, 0)),
        out_shape=jax.ShapeDtypeStruct((N, D), jnp.float32),
    )(partials)


@jax.jit
def _impl(x, edge_index, edge_weight, W, b):
    support = _linear(x, W, b)
    src = edge_index[0].astype(jnp.int32).reshape(NW, NST, NJS, B)
    dst = edge_index[1].astype(jnp.int32).reshape(NW, NST, NJS, B)
    zeros = jnp.zeros((NP, D), jnp.float32)
    partials = _sc_scatter(support, src, dst, edge_weight, zeros)
    return (partials[0, :N] + partials[1, :N])


def kernel(x, edge_index, edge_weight, W, b):
    return _impl(x, edge_index, edge_weight, W, b)


# speedup vs baseline: 1.0295x; 1.0086x over previous
import dataclasses

import jax
import jax.numpy as jnp
from jax import lax
from jax.experimental import pallas as pl
from jax.experimental.pallas import tpu as pltpu
from jax.experimental.pallas import tpu_sc as plsc

N, E, D = 10000, 320000, 128

NC = 2
NS = 16
NW = NC * NS
EPW = E // NW
NST = 10
B = 100
NJS = EPW // (NST * B)
EPS = NJS * B
RPS = 632
NP = NS * RPS


def _linear_body(x_ref, w_ref, b_ref, o_ref):
    o_ref[...] = lax.dot_general(
        x_ref[...], w_ref[...],
        dimension_numbers=(((1,), (1,)), ((), ())),
        preferred_element_type=jnp.float32,
    ) + b_ref[...]


def _linear(x, W, b):
    blk = 1000
    return pl.pallas_call(
        _linear_body,
        grid=(N // blk,),
        in_specs=[
            pl.BlockSpec((blk, D), lambda i: (i, 0)),
            pl.BlockSpec((D, D), lambda i: (0, 0)),
            pl.BlockSpec((1, D), lambda i: (0, 0)),
        ],
        out_specs=pl.BlockSpec((blk, D), lambda i: (i, 0)),
        out_shape=jax.ShapeDtypeStruct((N, D), jnp.float32),
    )(x, W, b.reshape(1, D))


def _sc_body(support_hbm, src_hbm, dst_hbm, w_hbm, zeros_hbm, out_hbm,
             src0, src1, dst0, dst1, w0, w1, gbuf0, gbuf1, gbuf2, acc,
             sg0, sg1, sg2, ss0, ss1, ss2, sem_st, sem_i):
    cid = lax.axis_index("c")
    sid = lax.axis_index("s")
    wid = sid * NC + cid
    srcbufs = (src0, src1)
    dstbufs = (dst0, dst1)
    wbufs = (w0, w1)
    gbufs = (gbuf0, gbuf1, gbuf2)
    gsems = (sg0, sg1, sg2)
    ssems = (ss0, ss1, ss2)

    def stage_in(s, bufset):
        pltpu.async_copy(src_hbm.at[wid, s], srcbufs[bufset], sem_st)
        pltpu.async_copy(dst_hbm.at[wid, s], dstbufs[bufset], sem_st)
        pltpu.async_copy(w_hbm.at[pl.ds(wid * EPW + s * EPS, EPS)],
                         wbufs[bufset], sem_st)

    def stage_wait(s, bufset):
        pltpu.make_async_copy(src_hbm.at[wid, s], srcbufs[bufset],
                              sem_st).wait()
        pltpu.make_async_copy(dst_hbm.at[wid, s], dstbufs[bufset],
                              sem_st).wait()
        pltpu.make_async_copy(w_hbm.at[pl.ds(wid * EPW + s * EPS, EPS)],
                              wbufs[bufset], sem_st).wait()

    stage_in(0, 0)
    pltpu.async_copy(zeros_hbm.at[pl.ds(sid * RPS, RPS)],
                     acc.at[pl.ds(sid * RPS, RPS)], sem_i).wait()
    plsc.subcore_barrier()
    stage_wait(0, 0)

    pltpu.async_copy(support_hbm.at[src0.at[0]], gbuf0, sg0)
    pltpu.async_copy(support_hbm.at[src0.at[1]], gbuf1, sg1)

    def _step(jj, m, srcbuf, dstbuf, wbuf):
        gbuf = gbufs[m]
        q = (m + 2) % 3
        pltpu.make_async_copy(support_hbm.at[srcbuf.at[jj]], gbuf,
                              gsems[m]).wait()

        @plsc.parallel_loop(0, B, unroll=4)
        def _(e):
            wsplat = plsc.load_gather(
                wbuf, [jnp.full((16,), jj * B + e, jnp.int32)])
            for c in range(D // 16):
                sl = pl.ds(c * 16, 16)
                gbuf[e, sl] = gbuf[e, sl] * wsplat

        pltpu.async_copy(gbuf, acc.at[dstbuf.at[jj]], ssems[m], add=True)

        @pl.when(jj + 2 < NJS)
        def _():
            @pl.when(jj >= 1)
            def _():
                pltpu.make_async_copy(gbufs[q], acc.at[dstbuf.at[jj]],
                                      ssems[q]).wait()
            pltpu.async_copy(support_hbm.at[srcbuf.at[jj + 2]],
                             gbufs[q], gsems[q])

    for s in range(NST):
        cs = s % 2
        srcbuf, dstbuf, wbuf = srcbufs[cs], dstbufs[cs], wbufs[cs]
        if s + 1 < NST:
            stage_in(s + 1, 1 - cs)

        @pl.loop(0, NJS - 1, step=3)
        def _(j):
            for m in range(3):
                _step(j + m, m, srcbuf, dstbuf, wbuf)

        _step(NJS - 1, (NJS - 1) % 3, srcbuf, dstbuf, wbuf)

        for i in range(3):
            jj = NJS - 3 + i
            pltpu.make_async_copy(gbufs[jj % 3], acc.at[dstbuf.at[jj]],
                                  ssems[jj % 3]).wait()

        if s + 1 < NST:
            stage_wait(s + 1, 1 - cs)
            nsrc = srcbufs[1 - cs]
            pltpu.async_copy(support_hbm.at[nsrc.at[0]], gbuf0, sg0)
            pltpu.async_copy(support_hbm.at[nsrc.at[1]], gbuf1, sg1)

    plsc.subcore_barrier()
    pltpu.sync_copy(acc.at[pl.ds(sid * RPS, RPS)],
                    out_hbm.at[cid, pl.ds(sid * RPS, RPS)])


def _sc_scatter(support, src, dst, w, zeros):
    mesh = plsc.VectorSubcoreMesh(core_axis_name="c", subcore_axis_name="s")
    cp = pltpu.CompilerParams()
    if "needs_layout_passes" in pltpu.CompilerParams.__dataclass_fields__:
        cp = dataclasses.replace(cp, needs_layout_passes=False)
    kern = pl.kernel(
        _sc_body,
        compiler_params=cp,
        out_type=jax.ShapeDtypeStruct((NC, NP, D), jnp.float32),
        mesh=mesh,
        scratch_types=[
            pltpu.VMEM((NJS, B), jnp.int32),
            pltpu.VMEM((NJS, B), jnp.int32),
            pltpu.VMEM((NJS, B), jnp.int32),
            pltpu.VMEM((NJS, B), jnp.int32),
            pltpu.VMEM((EPS,), jnp.float32),
            pltpu.VMEM((EPS,), jnp.float32),
            pltpu.VMEM((B, D), jnp.float32),
            pltpu.VMEM((B, D), jnp.float32),
            pltpu.VMEM((B, D), jnp.float32),
            pltpu.VMEM_SHARED((NP, D), jnp.float32),
            pltpu.SemaphoreType.DMA,
            pltpu.SemaphoreType.DMA,
            pltpu.SemaphoreType.DMA,
            pltpu.SemaphoreType.DMA,
            pltpu.SemaphoreType.DMA,
            pltpu.SemaphoreType.DMA,
            pltpu.SemaphoreType.DMA,
            pltpu.SemaphoreType.DMA,
        ],
    )
    return kern(support, src, dst, w, zeros)


def _combine_body(p_ref, o_ref):
    o_ref[...] = p_ref[0] + p_ref[1]


def _combine(partials):
    blk = 1000
    return pl.pallas_call(
        _combine_body,
        grid=(N // blk,),
        in_specs=[pl.BlockSpec((NC, blk, D), lambda i: (0, i, 0))],
        out_specs=pl.BlockSpec((blk, D), lambda i: (i, 0)),
        out_shape=jax.ShapeDtypeStruct((N, D), jnp.float32),
    )(partials)


@jax.jit
def _impl(x, edge_index, edge_weight, W, b):
    support = _linear(x, W, b)
    src = edge_index[0].astype(jnp.int32).reshape(NW, NST, NJS, B)
    dst = edge_index[1].astype(jnp.int32).reshape(NW, NST, NJS, B)
    zeros = jnp.zeros((NP, D), jnp.float32)
    partials = _sc_scatter(support, src, dst, edge_weight, zeros)
    return (partials[0, :N] + partials[1, :N])


def kernel(x, edge_index, edge_weight, W, b):
    return _impl(x, edge_index, edge_weight, W, b)
